# baseline (speedup 1.0000x reference)
import functools

import jax
import jax.numpy as jnp
from jax import lax
from jax.experimental import pallas as pl
from jax.experimental.pallas import tpu as pltpu
from jax.experimental.pallas import tpu_sc as plsc

_B = 16384
_RANK = 32
_SIGMA = 1.0
_LAMBD = 1.0 / 64.0

_NW = 32
_BPW = _B // _NW
_BLOCKS = _BPW // 16
_CHUNK = 128

_MANT = 0x007FFFFF
_ONEBITS = 0x3F800000
_SQRT2 = 1.41421356237
_LN2 = 0.6931471805599453

_LOGP = (7.0376836292e-2, -1.1514610310e-1, 1.1676998740e-1,
         -1.2420140846e-1, 1.4249322787e-1, -1.6668057665e-1,
         2.0000714765e-1, -2.4999993993e-1, 3.3333331174e-1)


_GDN = lax.GatherDimensionNumbers(
    offset_dims=(), collapsed_slice_dims=(0,), start_index_map=(0,))


def _permute(v, idx):
    return lax.gather(v, idx[:, None], dimension_numbers=_GDN,
                      slice_sizes=(1,),
                      mode=lax.GatherScatterMode.PROMISE_IN_BOUNDS)


def _hsum_all(v, lane):
    for k in (8, 4, 2, 1):
        v = v + _permute(v, lane ^ k)
    return v


def _full_log(x):
    iv = lax.bitcast_convert_type(x, jnp.int32)
    e = (iv >> 23) - 127
    m = lax.bitcast_convert_type((iv & _MANT) | _ONEBITS, jnp.float32)
    big = m > _SQRT2
    m = jnp.where(big, m * 0.5, m)
    e = jnp.where(big, e + 1, e)
    t = m - 1.0
    p = jnp.full((16,), _LOGP[0], jnp.float32)
    for c in _LOGP[1:]:
        p = p * t + c
    lnm = t + t * t * (t * p - 0.5)
    return lnm, e


def _sc_body(m0h, m1h, m2h, c0h, c1h, c2h, valsh, i0h, i1h, i2h,
             out_h,
             i0v, i1v, i2v, valsv,
             m0v, m1v, m2v, c0v, c1v, c2v,
             partv, sem):
    nc = 2
    wid = lax.axis_index("s") * nc + lax.axis_index("c")
    base = wid * _BPW

    pltpu.sync_copy(i0h.at[pl.ds(base, _BPW)], i0v)
    pltpu.sync_copy(i1h.at[pl.ds(base, _BPW)], i1v)
    pltpu.sync_copy(i2h.at[pl.ds(base, _BPW)], i2v)
    pltpu.sync_copy(valsh.at[pl.ds(base, _BPW)], valsv)

    copies = []
    for c in range(_BPW // _CHUNK):
        sl = pl.ds(c * _CHUNK, _CHUNK)
        for tab, idxv, dst in ((m0h, i0v, m0v), (m1h, i1v, m1v),
                               (m2h, i2v, m2v), (c0h, i0v, c0v),
                               (c1h, i1v, c1v), (c2h, i2v, c2v)):
            copies.append(pltpu.async_copy(tab.at[idxv.at[sl]], dst.at[sl], sem))
    for cp in copies:
        cp.wait()

    lane = lax.iota(jnp.int32, 16)
    zeros = jnp.zeros((16,), jnp.float32)

    def block(blk, carry):
        a_s1, a_m2, a_v, a_ln, a_e = carry
        predv = zeros
        pacc = jnp.full((16,), 1.0, jnp.float32)
        for i in range(16):
            r = blk * 16 + i
            m0a = m0v[r, pl.ds(0, 16)]
            m0b = m0v[r, pl.ds(16, 16)]
            m1a = m1v[r, pl.ds(0, 16)]
            m1b = m1v[r, pl.ds(16, 16)]
            m2a = m2v[r, pl.ds(0, 16)]
            m2b = m2v[r, pl.ds(16, 16)]
            ts = m0a * m1a * m2a + m0b * m1b * m2b
            predv = jnp.where(lane == i, _hsum_all(ts, lane), predv)
            a_m2 = (a_m2 + m0a * m0a + m0b * m0b + m1a * m1a
                    + m1b * m1b + m2a * m2a + m2b * m2b)
            for cv in (c0v, c1v, c2v):
                for off in (0, 16):
                    l = cv[r, pl.ds(off, 16)]
                    var = l * l
                    a_v = a_v + var
                    iv = lax.bitcast_convert_type(var, jnp.int32)
                    a_e = a_e + ((iv >> 23) - 127)
                    pacc = pacc * lax.bitcast_convert_type(
                        (iv & _MANT) | _ONEBITS, jnp.float32)
        q = valsv[pl.ds(blk * 16, 16)]
        d = q - predv
        a_s1 = a_s1 + d * d
        lnm, e = _full_log(pacc)
        return (a_s1, a_m2, a_v, a_ln + lnm, a_e + e)

    init = (zeros, zeros, zeros, zeros, jnp.zeros((16,), jnp.int32))
    a_s1, a_m2, a_v, a_ln, a_e = lax.fori_loop(0, _BLOCKS, block, init)

    partv[pl.ds(0, 16)] = a_s1
    partv[pl.ds(16, 16)] = a_m2
    partv[pl.ds(32, 16)] = a_v
    partv[pl.ds(48, 16)] = a_ln
    partv[pl.ds(64, 16)] = a_e.astype(jnp.float32)
    pltpu.sync_copy(partv, out_h.at[wid])


def _combine_body(parts_ref, o_ref):
    import math
    const = (0.5 * _B * math.log(2.0 * math.pi * _SIGMA ** 2)
             - 0.5 * _LAMBD * 3.0 * _B * _RANK)
    p = parts_ref[...]
    total = (0.5 / (_SIGMA ** 2) * jnp.sum(p[:, 0:16])
             + 0.5 * _LAMBD * jnp.sum(p[:, 16:32])
             + 0.5 * _LAMBD * jnp.sum(p[:, 32:48])
             - 0.5 * _LAMBD * jnp.sum(p[:, 48:64])
             - 0.5 * _LAMBD * _LN2 * jnp.sum(p[:, 64:80])
             + const)
    o_ref[...] = jnp.reshape(total, (1, 1))


def kernel(mean0, mean1, mean2, chol0, chol1, chol2, vals, idx0, idx1, idx2):
    mesh = plsc.VectorSubcoreMesh(core_axis_name="c", subcore_axis_name="s")
    sc = functools.partial(
        pl.kernel, mesh=mesh,
        compiler_params=pltpu.CompilerParams(use_tc_tiling_on_sc=False),
        out_type=jax.ShapeDtypeStruct((_NW, 80), jnp.float32),
        scratch_types=[
            pltpu.VMEM((_BPW,), jnp.int32),
            pltpu.VMEM((_BPW,), jnp.int32),
            pltpu.VMEM((_BPW,), jnp.int32),
            pltpu.VMEM((_BPW,), jnp.float32),
            pltpu.VMEM((_BPW, _RANK), jnp.float32),
            pltpu.VMEM((_BPW, _RANK), jnp.float32),
            pltpu.VMEM((_BPW, _RANK), jnp.float32),
            pltpu.VMEM((_BPW, _RANK), jnp.float32),
            pltpu.VMEM((_BPW, _RANK), jnp.float32),
            pltpu.VMEM((_BPW, _RANK), jnp.float32),
            pltpu.VMEM((80,), jnp.float32),
            pltpu.SemaphoreType.DMA,
        ],
    )(_sc_body)
    parts = sc(mean0, mean1, mean2, chol0, chol1, chol2,
               vals, idx0, idx1, idx2)
    out = pl.pallas_call(
        _combine_body,
        out_shape=jax.ShapeDtypeStruct((1, 1), jnp.float32),
    )(parts)
    return out[0, 0]
